# baseline (speedup 1.0000x reference)
import functools

import jax
import jax.numpy as jnp
from jax import lax
from jax.experimental import pallas as pl
from jax.experimental.pallas import tpu as pltpu
from jax.experimental.pallas import tpu_sc as plsc

B, T, D = 4, 2048, 128
N = B * T
NC, NS = 2, 16
NW = NC * NS
TW = T // NW
LG = D // 16

mesh = plsc.VectorSubcoreMesh(core_axis_name="c", subcore_axis_name="s")


@functools.partial(
    pl.kernel,
    mesh=mesh,
    out_type=jax.ShapeDtypeStruct((N, D), jnp.float32),
    scratch_types=[
        pltpu.VMEM((B, TW), jnp.int32),
        pltpu.VMEM((B * TW, D), jnp.float32),
        pltpu.VMEM((TW, D), jnp.float32),
        pltpu.SemaphoreType.DMA,
        pltpu.SemaphoreType.DMA,
        pltpu.SemaphoreType.DMA,
        pltpu.SemaphoreType.DMA,
        pltpu.SemaphoreType.DMA,
        pltpu.SemaphoreType.DMA,
    ],
)
def _emb_kernel(xr_hbm, tok_hbm, pos_hbm, out_hbm, idx_v, rows_v, pos_v,
                sem_p, sem_g0, sem_g1, sem_g2, sem_g3, sem_w):
    sem_g = [sem_g0, sem_g1, sem_g2, sem_g3]
    wid = lax.axis_index("s") * NC + lax.axis_index("c")
    tbase = wid * TW

    pltpu.sync_copy(xr_hbm.at[wid], idx_v)

    cp_pos = pltpu.async_copy(pos_hbm.at[pl.ds(tbase, TW)], pos_v, sem_p)

    gcps = [
        pltpu.async_copy(
            tok_hbm.at[idx_v.at[b]],
            rows_v.at[pl.ds(b * TW, TW)],
            sem_g[b],
        )
        for b in range(B)
    ]
    cp_pos.wait()

    wcps = []
    for b in range(B):
        gcps[b].wait()

        def body(t2, carry, b=b):
            for u in range(2):
                t = t2 * 2 + u
                r = b * TW + t
                for g in range(LG):
                    sl = pl.ds(g * 16, 16)
                    plsc.addupdate(rows_v.at[r, sl], pos_v[t, sl])
            return carry

        lax.fori_loop(0, TW // 2, body, 0)
        wcps.append(
            pltpu.async_copy(
                rows_v.at[pl.ds(b * TW, TW)],
                out_hbm.at[pl.ds(b * T + tbase, TW)],
                sem_w,
            )
        )
    for cp in wcps:
        cp.wait()


def kernel(x, tok_emb_table, pos_emb_table):
    xr = x.astype(jnp.int32).reshape(B, NW, TW).transpose(1, 0, 2)
    out = _emb_kernel(xr, tok_emb_table, pos_emb_table)
    return out.reshape(B, T, D)


# speedup vs baseline: 1.0467x; 1.0467x over previous
import functools

import jax
import jax.numpy as jnp
from jax import lax
from jax.experimental import pallas as pl
from jax.experimental.pallas import tpu as pltpu
from jax.experimental.pallas import tpu_sc as plsc

B, T, D = 4, 2048, 128
N = B * T
NC, NS = 2, 16
NW = NC * NS
TW = T // NW
LG = D // 16
HB = B // 2

mesh = plsc.VectorSubcoreMesh(core_axis_name="c", subcore_axis_name="s")


@functools.partial(
    pl.kernel,
    mesh=mesh,
    out_type=jax.ShapeDtypeStruct((N, D), jnp.float32),
    scratch_types=[
        pltpu.VMEM((2, HB * TW), jnp.int32),
        pltpu.VMEM((B * TW, D), jnp.float32),
        pltpu.VMEM((TW, D), jnp.float32),
        pltpu.SemaphoreType.DMA,
        pltpu.SemaphoreType.DMA,
        pltpu.SemaphoreType.DMA,
        pltpu.SemaphoreType.DMA,
        pltpu.SemaphoreType.DMA,
    ],
)
def _emb_kernel(xr_hbm, tok_hbm, pos_hbm, out_hbm, idx_v, rows_v, pos_v,
                sem_i, sem_p, sem_g0, sem_g1, sem_w):
    sem_g = [sem_g0, sem_g1]
    wid = lax.axis_index("s") * NC + lax.axis_index("c")
    tbase = wid * TW

    cp_i = pltpu.async_copy(xr_hbm.at[wid], idx_v, sem_i)
    cp_pos = pltpu.async_copy(pos_hbm.at[pl.ds(tbase, TW)], pos_v, sem_p)
    cp_i.wait()

    gcps = [
        pltpu.async_copy(
            tok_hbm.at[idx_v.at[h]],
            rows_v.at[pl.ds(h * HB * TW, HB * TW)],
            sem_g[h],
        )
        for h in range(2)
    ]
    cp_pos.wait()

    wcps = []
    for h in range(2):
        gcps[h].wait()

        def body(t, carry, h=h):
            for g in range(LG):
                sl = pl.ds(g * 16, 16)
                pv = pos_v[t, sl]
                for j in range(HB):
                    r = (h * HB + j) * TW + t
                    plsc.addupdate(rows_v.at[r, sl], pv)
            return carry

        lax.fori_loop(0, TW, body, 0)

        for j in range(HB):
            b = h * HB + j
            wcps.append(
                pltpu.async_copy(
                    rows_v.at[pl.ds(b * TW, TW)],
                    out_hbm.at[pl.ds(b * T + tbase, TW)],
                    sem_w,
                )
            )
    for cp in wcps:
        cp.wait()


def kernel(x, tok_emb_table, pos_emb_table):
    xr = x.astype(jnp.int32).reshape(B, NW, TW).transpose(1, 0, 2)
    xr = xr.reshape(NW, 2, HB * TW)
    out = _emb_kernel(xr, tok_emb_table, pos_emb_table)
    return out.reshape(B, T, D)
